# baseline (speedup 1.0000x reference)
import functools

import jax
import jax.numpy as jnp
from jax import lax
from jax.experimental import pallas as pl
from jax.experimental.pallas import tpu as pltpu
from jax.experimental.pallas import tpu_sc as plsc

BATCH = 16384
ENTITY_DIM = 64
RELATION_DIM = 64

_info = plsc.get_sparse_core_info()
_NC, _NS = _info.num_cores, _info.num_subcores
_NW = _NC * _NS
_BPW = BATCH // _NW

_mesh = plsc.VectorSubcoreMesh(core_axis_name="c", subcore_axis_name="s")


@functools.partial(
    pl.kernel,
    mesh=_mesh,
    compiler_params=pltpu.CompilerParams(use_tc_tiling_on_sc=False),
    out_type=[
        jax.ShapeDtypeStruct((BATCH, ENTITY_DIM), jnp.float32),
        jax.ShapeDtypeStruct((BATCH, RELATION_DIM), jnp.float32),
    ],
    scratch_types=[
        pltpu.VMEM((_BPW,), jnp.int32),
        pltpu.VMEM((_BPW, ENTITY_DIM), jnp.float32),
        pltpu.VMEM((_BPW,), jnp.int32),
        pltpu.VMEM((_BPW, RELATION_DIM), jnp.float32),
        pltpu.SemaphoreType.DMA,
        pltpu.SemaphoreType.DMA,
    ],
)
def _lookup(ent_hbm, e_hbm, rel_hbm, r_hbm, ent_out, rel_out,
            eidx_v, erows_v, ridx_v, rrows_v, sem_e, sem_r):
    wid = lax.axis_index("s") * _NC + lax.axis_index("c")
    base = wid * _BPW
    pltpu.sync_copy(e_hbm.at[pl.ds(base, _BPW)], eidx_v)
    pltpu.sync_copy(r_hbm.at[pl.ds(base, _BPW)], ridx_v)
    ce = pltpu.async_copy(ent_hbm.at[eidx_v], erows_v, sem_e)
    cr = pltpu.async_copy(rel_hbm.at[ridx_v], rrows_v, sem_r)
    ce.wait()
    pltpu.sync_copy(erows_v, ent_out.at[pl.ds(base, _BPW)])
    cr.wait()
    pltpu.sync_copy(rrows_v, rel_out.at[pl.ds(base, _BPW)])


def kernel(e, r, entity_table, relation_table):
    ent_emb, rel_emb = _lookup(
        entity_table,
        e.astype(jnp.int32),
        relation_table,
        r.astype(jnp.int32),
    )
    return (ent_emb, rel_emb)


# speedup vs baseline: 1.5871x; 1.5871x over previous
import functools

import jax
import jax.numpy as jnp
from jax import lax
from jax.experimental import pallas as pl
from jax.experimental.pallas import tpu as pltpu
from jax.experimental.pallas import tpu_sc as plsc

BATCH = 16384
EDIM = 64
N_ENT = 1000001
N_REL = 1001
_LAST_FULL_COL = N_ENT // 128 - 1
_TAIL_OFF = (_LAST_FULL_COL + 1) * 128
_TAIL_W = N_ENT - _TAIL_OFF

_info = plsc.get_sparse_core_info()
_NC, _NS = _info.num_cores, _info.num_subcores
_NW = _NC * _NS
_BPW = BATCH // _NW
_RING = 2

_mesh = plsc.VectorSubcoreMesh(core_axis_name="c", subcore_axis_name="s")

_i32 = jnp.int32


def _dpat(k):
    return lax.iota(_i32, 16) + (16 * k)


@functools.partial(
    pl.kernel,
    mesh=_mesh,
    compiler_params=pltpu.CompilerParams(
        use_tc_tiling_on_sc=True, needs_layout_passes=False),
    out_type=[
        jax.ShapeDtypeStruct((EDIM, BATCH), jnp.float32),
        jax.ShapeDtypeStruct((EDIM, BATCH), jnp.float32),
    ],
    scratch_types=[
        pltpu.SMEM((_BPW,), _i32),
        pltpu.SMEM((_BPW,), _i32),
        pltpu.VMEM((_BPW,), _i32),
        pltpu.VMEM((_BPW,), _i32),
        pltpu.VMEM((EDIM, 1024), jnp.float32),
        pltpu.VMEM((EDIM, _BPW), jnp.float32),
        pltpu.VMEM((EDIM, 128), jnp.float32),
        pltpu.VMEM((EDIM, 128), jnp.float32),
        pltpu.VMEM((EDIM, 128), jnp.float32),
        pltpu.SemaphoreType.DMA,
        pltpu.SemaphoreType.DMA,
    ],
)
def _lookup(et_hbm, e_hbm, rt_hbm, r_hbm, etail_hbm, eo_hbm, ro_hbm,
            eidx_s, ridx_s, eidx_v, ridx_v, relbuf, staging,
            rb0, rb1, tailbuf, sem0, sem1):
    wid = lax.axis_index("s") * _NC + lax.axis_index("c")
    base = wid * _BPW
    pltpu.sync_copy(e_hbm.at[pl.ds(base, _BPW)], eidx_v)
    pltpu.sync_copy(r_hbm.at[pl.ds(base, _BPW)], ridx_v)
    lane_i = lax.iota(_i32, 16)

    def spill_body(j, carry):
        ev = eidx_v[pl.ds(j * 16, 16)]
        rv = ridx_v[pl.ds(j * 16, 16)]
        for ll in range(16):
            eidx_s[j * 16 + ll] = jnp.sum(jnp.where(lane_i == ll, ev, 0))
            ridx_s[j * 16 + ll] = jnp.sum(jnp.where(lane_i == ll, rv, 0))
        return carry

    lax.fori_loop(0, _BPW // 16, spill_body, 0)

    rbufs = (rb0, rb1)
    sems = (sem0, sem1)

    pltpu.sync_copy(rt_hbm, relbuf)

    def rel_body(j, carry):
        rj = ridx_s[j]
        col = jnp.full((16,), j, _i32)
        ln = jnp.full((16,), rj, _i32)
        for k in range(4):
            v = plsc.load_gather(relbuf, [_dpat(k), ln])
            plsc.store_scatter(staging, [_dpat(k), col], v)
        return carry

    lax.fori_loop(0, _BPW, rel_body, 0)
    pltpu.sync_copy(staging, ro_hbm.at[:, pl.ds(base, _BPW)])

    def issue(j, k):
        ej = eidx_s[j]
        colc = jnp.minimum(ej >> 7, _LAST_FULL_COL + 1 - 1)
        off = pl.multiple_of(colc * 128, 128)
        pltpu.async_copy(et_hbm.at[:, pl.ds(off, 128)], rbufs[k], sems[k])

    for k in range(_RING):
        issue(k, k)

    def ent_body(o, carry):
        for k in range(_RING):
            i = o * _RING + k
            pltpu.make_async_copy(
                et_hbm.at[:, pl.ds(0, 128)], rbufs[k], sems[k]).wait()
            ei = eidx_s[i]
            colc = jnp.minimum(ei >> 7, _LAST_FULL_COL)
            ln_s = jnp.minimum(ei - colc * 128, 127)
            ln = jnp.full((16,), ln_s, _i32)
            col = jnp.full((16,), i, _i32)
            for kk in range(4):
                v = plsc.load_gather(rbufs[k], [_dpat(kk), ln])
                plsc.store_scatter(staging, [_dpat(kk), col], v)

            @pl.when(i + _RING < _BPW)
            def _():
                issue(i + _RING, k)
        return carry

    lax.fori_loop(0, _BPW // _RING, ent_body, 0)

    pltpu.sync_copy(etail_hbm, tailbuf)

    def tail_body(j, carry):
        ej = eidx_s[j]
        msk = jnp.full((16,), ej, _i32) >= _TAIL_OFF
        ln = jnp.full((16,), jnp.clip(ej - _TAIL_OFF, 0, _TAIL_W - 1), _i32)
        col = jnp.full((16,), j, _i32)
        for k in range(4):
            v = plsc.load_gather(tailbuf, [_dpat(k), ln])
            plsc.store_scatter(staging, [_dpat(k), col], v, mask=msk)
        return carry

    lax.fori_loop(0, _BPW, tail_body, 0)
    pltpu.sync_copy(staging, eo_hbm.at[:, pl.ds(base, _BPW)])


def kernel(e, r, entity_table, relation_table):
    rt_pad = jnp.pad(relation_table, ((0, 1024 - N_REL), (0, 0)))
    etail = jnp.pad(entity_table[_TAIL_OFF:], ((0, 128 - _TAIL_W), (0, 0)))
    eo, ro = _lookup(
        entity_table.T,
        e.astype(jnp.int32),
        rt_pad.T,
        r.astype(jnp.int32),
        etail.T,
    )
    return (eo.T, ro.T)


# speedup vs baseline: 2.1708x; 1.3678x over previous
import functools

import jax
import jax.numpy as jnp
from jax import lax
from jax.experimental import pallas as pl
from jax.experimental.pallas import tpu as pltpu
from jax.experimental.pallas import tpu_sc as plsc

BATCH = 16384
EDIM = 64
N_ENT = 1000001
N_REL = 1001
_TAIL_OFF = (N_ENT // 128) * 128
_TAIL_W = N_ENT - _TAIL_OFF
_NCOLS = N_ENT // 128

_info = plsc.get_sparse_core_info()
_NC, _NS = _info.num_cores, _info.num_subcores
_NW = _NC * _NS
_BPW = BATCH // _NW
_CPW = -(-_NCOLS // _NW)
_G = 2
_NSB = -(-_CPW // _G)
_SCROWS = BATCH + 128

_mesh = plsc.VectorSubcoreMesh(core_axis_name="c", subcore_axis_name="s")

_i32 = jnp.int32


def _dpat(k):
    return lax.iota(_i32, 16) + (16 * k)


def _lane_extract(vec, ll):
    return jnp.sum(jnp.where(lax.iota(_i32, 16) == ll, vec, 0))


@functools.partial(
    pl.kernel,
    mesh=_mesh,
    compiler_params=pltpu.CompilerParams(
        use_tc_tiling_on_sc=True, needs_layout_passes=False),
    out_type=[jax.ShapeDtypeStruct((_SCROWS, 128), jnp.float32)],
    scratch_types=[
        pltpu.VMEM((2048,), _i32),
        pltpu.VMEM((BATCH,), _i32),
        pltpu.VMEM((BATCH,), _i32),
        pltpu.VMEM((BATCH,), _i32),
        pltpu.VMEM((BATCH,), _i32),
        pltpu.VMEM((EDIM, _G * 128), jnp.float32),
        pltpu.VMEM((EDIM, _G * 128), jnp.float32),
        pltpu.VMEM((128, 128), jnp.float32),
        pltpu.VMEM((128,), _i32),
        pltpu.SemaphoreType.DMA,
        pltpu.SemaphoreType.DMA,
        pltpu.SemaphoreType.DMA,
    ],
)
def _phase1(et_hbm, e_hbm, scr_hbm,
            e_all, wl_e, wl_b, cur_e, cur_b, sb0, sb1, scat, bx,
            sem0, sem1, sem2):
    wid = lax.axis_index("s") * _NC + lax.axis_index("c")
    c_lo = wid * _CPW
    c_hi = jnp.minimum(c_lo + _CPW, _NCOLS)
    lane_i = lax.iota(_i32, 16)

    def chunk_body(q, off):
        pltpu.sync_copy(e_hbm.at[pl.ds(q * 2048, 2048)], e_all)

        def scan_body(j, off2):
            ev = e_all[pl.ds(j * 16, 16)]
            cv = lax.shift_right_logical(ev, 7)
            m = (cv >= c_lo) & (cv < c_hi)
            bv = lane_i + (q * 2048 + j * 16)
            plsc.store_compressed(wl_e.at[pl.ds(off2, 16)], ev, mask=m)
            plsc.store_compressed(wl_b.at[pl.ds(off2, 16)], bv, mask=m)
            return off2 + jnp.sum(m.astype(_i32))

        return lax.fori_loop(0, 128, scan_body, off)

    n = lax.fori_loop(0, BATCH // 2048, chunk_body, 0)
    nblk = lax.div(n + 15, 16)

    def bx_init(g, carry):
        bx[pl.ds(g * 16, 16)] = lane_i + (BATCH + g * 16)
        return carry

    lax.fori_loop(0, 8, bx_init, 0)

    sbufs = (sb0, sb1)
    sems = (sem0, sem1)

    def woff_of(sb):
        c0 = c_lo + sb * _G
        return pl.multiple_of(
            jnp.minimum(c0 * 128, _TAIL_OFF - _G * 128), 128)

    def issue(sb, k):
        pltpu.async_copy(
            et_hbm.at[:, pl.ds(woff_of(sb), _G * 128)], sbufs[k], sems[k])

    for k in range(2):
        issue(k, k)

    def flush(slot):
        pltpu.async_copy(scat, scr_hbm.at[bx], sem2).wait()
        lax.fori_loop(0, 8, bx_init, 0)
        return slot * 0

    def sb_body(sb, slot):
        k0 = lax.rem(sb, 2)
        for k in range(2):

            @pl.when(k0 == k)
            def _():
                pltpu.make_async_copy(
                    et_hbm.at[:, pl.ds(0, _G * 128)], sbufs[k], sems[k]).wait()

        woff = woff_of(sb)
        c0 = c_lo + sb * _G

        def sel_body(j, off):
            ev = wl_e[pl.ds(j * 16, 16)]
            bv = wl_b[pl.ds(j * 16, 16)]
            cv = lax.shift_right_logical(ev, 7)
            inblk = (cv >= c0) & (cv < c0 + _G)
            m = inblk & (lane_i + j * 16 < n)
            plsc.store_compressed(cur_e.at[pl.ds(off, 16)], ev, mask=m)
            plsc.store_compressed(cur_b.at[pl.ds(off, 16)], bv, mask=m)
            return off + jnp.sum(m.astype(_i32))

        nsel = lax.fori_loop(0, nblk, sel_body, 0)

        def ent_body(t, slot_c):
            grp = lax.div(t, 16)
            ev = cur_e[pl.ds(grp * 16, 16)]
            bv = cur_b[pl.ds(grp * 16, 16)]
            lsel = lax.rem(t, 16)
            ei = jnp.sum(jnp.where(lane_i == lsel, ev, 0))
            bi = jnp.sum(jnp.where(lane_i == lsel, bv, 0))
            pos = jnp.full((16,), jnp.clip(ei - woff, 0, _G * 128 - 1), _i32)
            row = jnp.full((16,), slot_c, _i32)
            use0 = jnp.full((16,), k0, _i32) == 0
            for kk in range(4):
                v0 = plsc.load_gather(sb0, [_dpat(kk), pos])
                v1 = plsc.load_gather(sb1, [_dpat(kk), pos])
                plsc.store_scatter(scat, [row, _dpat(kk)],
                                   jnp.where(use0, v0, v1))

            plsc.store_scatter(bx, [row], jnp.full((16,), bi, _i32),
                               mask=lane_i == 0)
            slot_c = slot_c + 1
            slot_c = lax.cond(slot_c == 128, flush, lambda s: s, slot_c)
            return slot_c

        slot = lax.fori_loop(0, nsel, ent_body, slot)

        for k in range(2):

            @pl.when((k0 == k) & (sb + 2 < _NSB))
            def _(k=k):
                issue(sb + 2, k)

        return slot

    slot = lax.fori_loop(0, _NSB, sb_body, 0)

    @pl.when(slot > 0)
    def _():
        pltpu.async_copy(scat, scr_hbm.at[bx], sem2).wait()


@functools.partial(
    pl.kernel,
    mesh=_mesh,
    compiler_params=pltpu.CompilerParams(
        use_tc_tiling_on_sc=True, needs_layout_passes=False),
    out_type=[
        jax.ShapeDtypeStruct((EDIM, BATCH), jnp.float32),
        jax.ShapeDtypeStruct((EDIM, BATCH), jnp.float32),
    ],
    scratch_types=[
        pltpu.SMEM((_BPW,), _i32),
        pltpu.SMEM((_BPW,), _i32),
        pltpu.VMEM((_BPW,), _i32),
        pltpu.VMEM((_BPW,), _i32),
        pltpu.VMEM((EDIM, 1024), jnp.float32),
        pltpu.VMEM((EDIM, _BPW), jnp.float32),
        pltpu.VMEM((128, 128), jnp.float32),
        pltpu.VMEM((EDIM, 128), jnp.float32),
    ],
)
def _phase2(scr_hbm, e_hbm, r_hbm, rt_hbm, etail_hbm, eo_hbm, ro_hbm,
            eidx_s, ridx_s, eidx_v, ridx_v, relbuf, staging, chunk, tailbuf):
    wid = lax.axis_index("s") * _NC + lax.axis_index("c")
    base = wid * _BPW
    lane_i = lax.iota(_i32, 16)

    pltpu.sync_copy(e_hbm.at[pl.ds(base, _BPW)], eidx_v)
    pltpu.sync_copy(r_hbm.at[pl.ds(base, _BPW)], ridx_v)

    def spill_body(j, carry):
        ev = eidx_v[pl.ds(j * 16, 16)]
        rv = ridx_v[pl.ds(j * 16, 16)]
        for ll in range(16):
            eidx_s[j * 16 + ll] = _lane_extract(ev, ll)
            ridx_s[j * 16 + ll] = _lane_extract(rv, ll)
        return carry

    lax.fori_loop(0, _BPW // 16, spill_body, 0)

    pltpu.sync_copy(rt_hbm, relbuf)

    def rel_body(j, carry):
        rj = ridx_s[j]
        col = jnp.full((16,), j, _i32)
        ln = jnp.full((16,), rj, _i32)
        for k in range(4):
            v = plsc.load_gather(relbuf, [_dpat(k), ln])
            plsc.store_scatter(staging, [_dpat(k), col], v)
        return carry

    lax.fori_loop(0, _BPW, rel_body, 0)
    pltpu.sync_copy(staging, ro_hbm.at[:, pl.ds(base, _BPW)])

    for q in range(_BPW // 128):
        pltpu.sync_copy(scr_hbm.at[pl.ds(base + q * 128, 128), :], chunk)

        def tr_body(g, carry):
            rows = lane_i + g * 16
            for d in range(EDIM):
                dv = jnp.full((16,), d, _i32)
                v = plsc.load_gather(chunk, [rows, dv])
                plsc.store_scatter(staging, [dv, rows + q * 128], v)
            return carry

        lax.fori_loop(0, 8, tr_body, 0)

    pltpu.sync_copy(etail_hbm, tailbuf)

    def tail_body(j, carry):
        ej = eidx_s[j]
        msk = jnp.full((16,), ej, _i32) >= _TAIL_OFF
        ln = jnp.full((16,), jnp.clip(ej - _TAIL_OFF, 0, _TAIL_W - 1), _i32)
        col = jnp.full((16,), j, _i32)
        for k in range(4):
            v = plsc.load_gather(tailbuf, [_dpat(k), ln])
            plsc.store_scatter(staging, [_dpat(k), col], v, mask=msk)
        return carry

    lax.fori_loop(0, _BPW, tail_body, 0)
    pltpu.sync_copy(staging, eo_hbm.at[:, pl.ds(base, _BPW)])


def kernel(e, r, entity_table, relation_table):
    e32 = e.astype(jnp.int32)
    rt_pad = jnp.pad(relation_table, ((0, 1024 - N_REL), (0, 0)))
    etail = jnp.pad(entity_table[_TAIL_OFF:], ((0, 128 - _TAIL_W), (0, 0)))
    (scratch,) = _phase1(entity_table.T, e32)
    eo, ro = _phase2(scratch, e32, r.astype(jnp.int32), rt_pad.T, etail.T)
    return (eo.T, ro.T)
